# baseline (speedup 1.0000x reference)
import functools

import jax
import jax.numpy as jnp
from jax import lax
from jax.experimental import pallas as pl
from jax.experimental.pallas import tpu as pltpu
from jax.experimental.pallas import tpu_sc as plsc

EPS = 1e-5
K = 32
R2 = 0.3 * 0.3
N = 4096
NK = N * K
NW = 32


def _sel_body(pb_ref, pt_ref, out_ref, d_ref):
    pb = pb_ref[...]
    pt = pt_ref[...]
    srow = jnp.sum(pb * pb, axis=1, keepdims=True)
    scol = jnp.sum(pt * pt, axis=0, keepdims=True)
    D = srow + scol - 2.0 * jnp.dot(pb, pt, preferred_element_type=jnp.float32)

    iot = lax.broadcasted_iota(jnp.int32, (128, N), 1)
    col = lax.broadcasted_iota(jnp.int32, (128, 128), 1)

    mask = D <= R2
    cnt = mask.astype(jnp.int32)
    sh = 1
    while sh < N:
        cnt = cnt + jnp.concatenate(
            [jnp.zeros((128, sh), jnp.int32), cnt[:, : N - sh]], axis=1)
        sh *= 2
    ball0 = jnp.min(jnp.where(mask & (cnt == 1), iot, N), axis=1)
    res = jnp.where(col == 64, ball0[:, None], jnp.zeros((128, 128), jnp.int32))

    def ball_step(s, res):
        cand = jnp.where(mask & (cnt == s + 1), iot, N)
        idx = jnp.min(cand, axis=1)
        idx = jnp.where(idx == N, ball0, idx)
        return jnp.where(col == 64 + s, idx[:, None], res)

    res = lax.fori_loop(1, K, ball_step, res)

    d_ref[...] = D

    def topk_step(t, res):
        Dw = d_ref[...]
        m = jnp.min(Dw, axis=1, keepdims=True)
        arg = jnp.min(jnp.where(Dw == m, iot, N), axis=1)
        d_ref[...] = jnp.where(iot == arg[:, None], jnp.float32(jnp.inf), Dw)
        return jnp.where(col == t, arg[:, None], res)

    res = lax.fori_loop(0, K + 1, topk_step, res)
    out_ref[...] = res


def _select(ppad, ppad_t):
    return pl.pallas_call(
        _sel_body,
        grid=(N // 128,),
        in_specs=[
            pl.BlockSpec((128, 8), lambda i: (i, 0)),
            pl.BlockSpec((8, N), lambda i: (0, 0)),
        ],
        out_specs=pl.BlockSpec((128, 128), lambda i: (i, 0)),
        out_shape=jax.ShapeDtypeStruct((N, 128), jnp.int32),
        scratch_shapes=[pltpu.VMEM((128, N), jnp.float32)],
    )(ppad, ppad_t)


def _sc_ec_reduce(y, idxflat, z, use_spmem=True):
    C = y.shape[1]
    nodes_pw = N // NW
    rows_pw = nodes_pw * K
    CH = 8 if C <= 128 else 4
    chunk = CH * K
    nch = nodes_pw // CH
    NCH = C // 16
    mesh = plsc.VectorSubcoreMesh(core_axis_name="c", subcore_axis_name="s")

    @functools.partial(
        pl.kernel,
        mesh=mesh,
        out_type=[
            jax.ShapeDtypeStruct((N, C), jnp.float32),
            jax.ShapeDtypeStruct((NW, 8, C), jnp.float32),
        ],
        scratch_types=[
            pltpu.VMEM((rows_pw,), jnp.int32),
            pltpu.VMEM((2, chunk, C), jnp.float32),
            pltpu.VMEM((nodes_pw, C), jnp.float32),
            pltpu.VMEM((CH, C), jnp.float32),
            pltpu.VMEM((8, C), jnp.float32),
            (pltpu.VMEM_SHARED((N, C), jnp.float32) if use_spmem
             else pltpu.VMEM((8,), jnp.float32)),
            pltpu.SemaphoreType.DMA,
        ],
    )
    def k(y_hbm, idx_hbm, z_hbm, m_hbm, part_hbm, idx_v, rows_v, m_v, z_v,
          p_v, ysh, gsem):
        wid = lax.axis_index("s") * 2 + lax.axis_index("c")
        base_row = wid * rows_pw
        base_node = wid * nodes_pw
        if use_spmem:
            ytab = ysh
            sid = lax.axis_index("s")
            stage = N // 16
            so = pl.multiple_of(sid * stage, stage)
            pltpu.sync_copy(y_hbm.at[pl.ds(so, stage)],
                            ysh.at[pl.ds(so, stage)])
        else:
            ytab = y_hbm
        pltpu.sync_copy(idx_hbm.at[pl.ds(base_row, rows_pw)], idx_v)
        if use_spmem:
            plsc.subcore_barrier()
        for cc in range(NCH):
            zv = jnp.zeros((16,), jnp.float32)
            p_v[0, pl.ds(cc * 16, 16)] = zv
            p_v[1, pl.ds(cc * 16, 16)] = zv

        def gstart(j, b):
            off = pl.multiple_of(j * chunk, chunk)
            pltpu.async_copy(
                ytab.at[idx_v.at[pl.ds(off, chunk)]], rows_v.at[b], gsem)

        def gwait():
            pltpu.make_async_copy(
                ytab.at[idx_v.at[pl.ds(0, chunk)]], rows_v.at[0], gsem
            ).wait()

        gstart(0, 0)

        def outer(jh, _):
            for b in range(2):
                j = jh * 2 + b

                @pl.when(j + 1 < nch)
                def _():
                    gstart(j + 1, 1 - b)

                pltpu.sync_copy(
                    z_hbm.at[pl.ds(base_node + j * CH, CH)], z_v)
                gwait()
                for i in range(CH):
                    r0 = i * K
                    for cc in range(NCH):
                        c0 = cc * 16
                        ga0 = rows_v[b, r0, pl.ds(c0, 16)]
                        gb0 = rows_v[b, r0 + K // 2, pl.ds(c0, 16)]

                        def kstep(kk, acc):
                            ga = rows_v[b, r0 + kk, pl.ds(c0, 16)]
                            gb = rows_v[b, r0 + K // 2 + kk, pl.ds(c0, 16)]
                            return (jnp.maximum(acc[0], ga),
                                    jnp.maximum(acc[1], gb),
                                    acc[2] + (ga + gb),
                                    acc[3] + (ga * ga + gb * gb))

                        mxa, mxb, s1, s2 = lax.fori_loop(
                            1, K // 2, kstep,
                            (ga0, gb0, ga0 + gb0, ga0 * ga0 + gb0 * gb0))
                        mx = jnp.maximum(mxa, mxb)
                        m_v[j * CH + i, pl.ds(c0, 16)] = mx
                        zr = z_v[i, pl.ds(c0, 16)]
                        p_v[0, pl.ds(c0, 16)] += K * zr + s1
                        p_v[1, pl.ds(c0, 16)] += (
                            K * zr * zr + 2.0 * zr * s1 + s2)
            return 0

        lax.fori_loop(0, nch // 2, outer, 0)
        pltpu.sync_copy(m_v, m_hbm.at[pl.ds(base_node, nodes_pw)])
        pltpu.sync_copy(p_v, part_hbm.at[wid])

    return k(y, idxflat, z)


def _sc_ppf_gather(tbl8, ballflat):
    rows_pw = NK // NW
    mesh = plsc.VectorSubcoreMesh(core_axis_name="c", subcore_axis_name="s")

    chunk = 256
    nch = rows_pw // chunk

    @functools.partial(
        pl.kernel,
        mesh=mesh,
        out_type=jax.ShapeDtypeStruct((NK, 128), jnp.float32),
        scratch_types=[
            pltpu.VMEM((rows_pw,), jnp.int32),
            pltpu.VMEM((2, chunk, 128), jnp.float32),
            pltpu.VMEM_SHARED((N, 128), jnp.float32),
            pltpu.SemaphoreType.DMA,
        ],
    )
    def k(tbl_hbm, idx_hbm, out_hbm, idx_v, rows_v, sh, sem):
        sid = lax.axis_index("s")
        wid = sid * 2 + lax.axis_index("c")
        base = wid * rows_pw
        stage = N // 16
        so = pl.multiple_of(sid * stage, stage)
        pltpu.sync_copy(tbl_hbm.at[pl.ds(so, stage)], sh.at[pl.ds(so, stage)])
        pltpu.sync_copy(idx_hbm.at[pl.ds(base, rows_pw)], idx_v)
        plsc.subcore_barrier()

        def gstart(j, b):
            off = pl.multiple_of(j * chunk, chunk)
            pltpu.async_copy(
                sh.at[idx_v.at[pl.ds(off, chunk)]], rows_v.at[b], sem)

        def gwait():
            pltpu.make_async_copy(
                sh.at[idx_v.at[pl.ds(0, chunk)]], rows_v.at[0],
                sem).wait()

        gstart(0, 0)
        for j in range(nch):
            if j + 1 < nch:
                gstart(j + 1, (j + 1) % 2)
            gwait()
            pltpu.sync_copy(rows_v.at[j % 2],
                            out_hbm.at[pl.ds(base + j * chunk, chunk)])

    return k(tbl8, ballflat)


def _ppf_ang_body(gp_ref, pt_ref, nt_ref, w0_ref, f10_ref, st_ref):
    kstep = pl.program_id(0)
    g = gp_ref[...]
    i8 = lax.broadcasted_iota(jnp.int32, (8, 128), 0)
    j8 = lax.broadcasted_iota(jnp.int32, (8, 128), 1)
    eye8 = (i8 == j8).astype(jnp.float32)
    gp = lax.dot_general(eye8, g, (((1,), (1,)), ((), ())),
                         preferred_element_type=jnp.float32)
    px, py, pz = pt_ref[0:1, :], pt_ref[1:2, :], pt_ref[2:3, :]
    nix, niy, niz = nt_ref[0:1, :], nt_ref[1:2, :], nt_ref[2:3, :]
    gx = gp[0:1, :] - px
    gy = gp[1:2, :] - py
    gz = gp[2:3, :] - pz
    njx, njy, njz = gp[3:4, :], gp[4:5, :], gp[5:6, :]

    def ang(ax, ay, az, bx, by, bz):
        cx = ay * bz - az * by
        cy = az * bx - ax * bz
        cz = ax * by - ay * bx
        yv = jnp.sqrt(cx * cx + cy * cy + cz * cz + 1e-12)
        xv = ax * bx + ay * by + az * bz
        return jnp.arctan2(yv, xv)

    nr_d = ang(nix, niy, niz, gx, gy, gz)
    ni_d = ang(njx, njy, njz, gx, gy, gz)
    nr_ni = ang(nix, niy, niz, njx, njy, njz)
    dn = jnp.sqrt(gx * gx + gy * gy + gz * gz + 1e-12)
    zr = jnp.zeros((6, N), jnp.float32)
    f10 = jnp.concatenate(
        [px, py, pz, gx, gy, gz, nr_d, ni_d, nr_ni, dn, zr], axis=0)
    f10_ref[...] = f10

    x0 = jnp.dot(w0_ref[...], f10, preferred_element_type=jnp.float32)

    @pl.when(kstep == 0)
    def _():
        st_ref[...] = jnp.zeros_like(st_ref)

    st_ref[0:1, 0:64] += jnp.sum(x0, axis=1)[None, :]
    st_ref[1:2, 0:64] += jnp.sum(x0 * x0, axis=1)[None, :]


def _ppf_ang(gplanes, pt, nt, w0p):
    return pl.pallas_call(
        _ppf_ang_body,
        grid=(K,),
        in_specs=[
            pl.BlockSpec((N, 128), lambda k: (k, 0)),
            pl.BlockSpec((4, N), lambda k: (0, 0)),
            pl.BlockSpec((4, N), lambda k: (0, 0)),
            pl.BlockSpec((64, 16), lambda k: (0, 0)),
        ],
        out_specs=[
            pl.BlockSpec((16, N), lambda k: (0, k)),
            pl.BlockSpec((8, 128), lambda k: (0, 0)),
        ],
        out_shape=[
            jax.ShapeDtypeStruct((16, NK), jnp.float32),
            jax.ShapeDtypeStruct((8, 128), jnp.float32),
        ],
    )(gplanes, pt, nt, w0p)


def _ppf_mid_body(f10_ref, w0_ref, w1_ref, m0_ref, i0_ref, st_ref):
    kstep = pl.program_id(0)
    x0 = jnp.dot(w0_ref[...], f10_ref[...], preferred_element_type=jnp.float32)
    h0 = (x0 - m0_ref[...]) * i0_ref[...]
    h0 = jnp.maximum(h0, 0.0)
    x1 = jnp.dot(w1_ref[...], h0, preferred_element_type=jnp.float32)

    @pl.when(kstep == 0)
    def _():
        st_ref[...] = jnp.zeros_like(st_ref)

    st_ref[0:1, :] += jnp.sum(x1, axis=1)[None, :]
    st_ref[1:2, :] += jnp.sum(x1 * x1, axis=1)[None, :]


def _ppf_mid(f10, w0p, w1, m0, i0):
    return pl.pallas_call(
        _ppf_mid_body,
        grid=(K,),
        in_specs=[
            pl.BlockSpec((16, N), lambda k: (0, k)),
            pl.BlockSpec((64, 16), lambda k: (0, 0)),
            pl.BlockSpec((128, 64), lambda k: (0, 0)),
            pl.BlockSpec((64, 1), lambda k: (0, 0)),
            pl.BlockSpec((64, 1), lambda k: (0, 0)),
        ],
        out_specs=pl.BlockSpec((8, 128), lambda k: (0, 0)),
        out_shape=jax.ShapeDtypeStruct((8, 128), jnp.float32),
    )(f10, w0p, w1, m0, i0)


def _ppf_last_body(f10_ref, w0_ref, w1_ref, w2_ref, m0_ref, i0_ref,
                   m1_ref, i1_ref, xmax_ref, st_ref):
    kstep = pl.program_id(0)
    x0 = jnp.dot(w0_ref[...], f10_ref[...], preferred_element_type=jnp.float32)
    h0 = jnp.maximum((x0 - m0_ref[...]) * i0_ref[...], 0.0)
    x1 = jnp.dot(w1_ref[...], h0, preferred_element_type=jnp.float32)
    h1 = jnp.maximum((x1 - m1_ref[...]) * i1_ref[...], 0.0)
    x2 = jnp.dot(w2_ref[...], h1, preferred_element_type=jnp.float32)

    @pl.when(kstep == 0)
    def _():
        xmax_ref[...] = x2
        st_ref[...] = jnp.zeros_like(st_ref)

    @pl.when(kstep > 0)
    def _():
        xmax_ref[...] = jnp.maximum(xmax_ref[...], x2)

    st_ref[0:1, 0:64] += jnp.sum(x2, axis=1)[None, :]
    st_ref[1:2, 0:64] += jnp.sum(x2 * x2, axis=1)[None, :]


def _ppf_last(f10, w0p, w1, w2, m0, i0, m1, i1):
    return pl.pallas_call(
        _ppf_last_body,
        grid=(K,),
        in_specs=[
            pl.BlockSpec((16, N), lambda k: (0, k)),
            pl.BlockSpec((64, 16), lambda k: (0, 0)),
            pl.BlockSpec((128, 64), lambda k: (0, 0)),
            pl.BlockSpec((64, 128), lambda k: (0, 0)),
            pl.BlockSpec((64, 1), lambda k: (0, 0)),
            pl.BlockSpec((64, 1), lambda k: (0, 0)),
            pl.BlockSpec((128, 1), lambda k: (0, 0)),
            pl.BlockSpec((128, 1), lambda k: (0, 0)),
        ],
        out_specs=[
            pl.BlockSpec((64, N), lambda k: (0, 0)),
            pl.BlockSpec((8, 128), lambda k: (0, 0)),
        ],
        out_shape=[
            jax.ShapeDtypeStruct((64, N), jnp.float32),
            jax.ShapeDtypeStruct((8, 128), jnp.float32),
        ],
    )(f10, w0p, w1, w2, m0, i0, m1, i1)


def _mm2_body(f_ref, wa_ref, wb_ref, ya_ref, yb_ref):
    f = f_ref[...]
    ya_ref[...] = jnp.dot(f, wa_ref[...], preferred_element_type=jnp.float32)
    yb_ref[...] = jnp.dot(f, wb_ref[...], preferred_element_type=jnp.float32)


def _mm2(f, wa, wb):
    Cin = f.shape[1]
    Ca, Cb = wa.shape[1], wb.shape[1]
    return pl.pallas_call(
        _mm2_body,
        grid=(N // 512,),
        in_specs=[
            pl.BlockSpec((512, Cin), lambda i: (i, 0)),
            pl.BlockSpec((Cin, Ca), lambda i: (0, 0)),
            pl.BlockSpec((Cin, Cb), lambda i: (0, 0)),
        ],
        out_specs=[
            pl.BlockSpec((512, Ca), lambda i: (i, 0)),
            pl.BlockSpec((512, Cb), lambda i: (i, 0)),
        ],
        out_shape=[
            jax.ShapeDtypeStruct((N, Ca), jnp.float32),
            jax.ShapeDtypeStruct((N, Cb), jnp.float32),
        ],
    )(f, wa, wb)


def _lrelu(x):
    return jnp.where(x >= 0, x, 0.2 * x)


def _gcnb_body(z_ref, mx_ref, mu_ref, iv_ref, wa_ref, wb_ref,
               f1_ref, ya_ref, yb_ref):
    f1 = _lrelu((z_ref[...] + mx_ref[...] - mu_ref[...]) * iv_ref[...])
    f1_ref[...] = f1
    ya_ref[...] = jnp.dot(f1, wa_ref[...], preferred_element_type=jnp.float32)
    yb_ref[...] = jnp.dot(f1, wb_ref[...], preferred_element_type=jnp.float32)


def _gcnb(z, mx, mu, iv, wa, wb):
    Cin = z.shape[1]
    Ca, Cb = wa.shape[1], wb.shape[1]
    return pl.pallas_call(
        _gcnb_body,
        grid=(N // 512,),
        in_specs=[
            pl.BlockSpec((512, Cin), lambda i: (i, 0)),
            pl.BlockSpec((512, Cin), lambda i: (i, 0)),
            pl.BlockSpec((1, Cin), lambda i: (0, 0)),
            pl.BlockSpec((1, Cin), lambda i: (0, 0)),
            pl.BlockSpec((Cin, Ca), lambda i: (0, 0)),
            pl.BlockSpec((Cin, Cb), lambda i: (0, 0)),
        ],
        out_specs=[
            pl.BlockSpec((512, Cin), lambda i: (i, 0)),
            pl.BlockSpec((512, Ca), lambda i: (i, 0)),
            pl.BlockSpec((512, Cb), lambda i: (i, 0)),
        ],
        out_shape=[
            jax.ShapeDtypeStruct((N, Cin), jnp.float32),
            jax.ShapeDtypeStruct((N, Ca), jnp.float32),
            jax.ShapeDtypeStruct((N, Cb), jnp.float32),
        ],
    )(z, mx, mu, iv, wa, wb)


def _gcnc_body(f_ref, f1_ref, z2_ref, mx2_ref, mu_ref, iv_ref, w3_ref,
               g3_ref, st_ref):
    i = pl.program_id(0)
    f2 = _lrelu((z2_ref[...] + mx2_ref[...] - mu_ref[...]) * iv_ref[...])
    f3 = jnp.concatenate([f_ref[...], f1_ref[...], f2], axis=1)
    g3 = jnp.dot(f3, w3_ref[...], preferred_element_type=jnp.float32)
    g3_ref[...] = g3

    @pl.when(i == 0)
    def _():
        st_ref[...] = jnp.zeros_like(st_ref)

    st_ref[0:1, :] += jnp.sum(g3, axis=0)[None, :]
    st_ref[1:2, :] += jnp.sum(g3 * g3, axis=0)[None, :]


def _gcnc(f, f1, z2, mx2, mu, iv, w3t):
    return pl.pallas_call(
        _gcnc_body,
        grid=(N // 512,),
        in_specs=[
            pl.BlockSpec((512, 128), lambda i: (i, 0)),
            pl.BlockSpec((512, 128), lambda i: (i, 0)),
            pl.BlockSpec((512, 256), lambda i: (i, 0)),
            pl.BlockSpec((512, 256), lambda i: (i, 0)),
            pl.BlockSpec((1, 256), lambda i: (0, 0)),
            pl.BlockSpec((1, 256), lambda i: (0, 0)),
            pl.BlockSpec((512, 128), lambda i: (0, 0)),
        ],
        out_specs=[
            pl.BlockSpec((512, 128), lambda i: (i, 0)),
            pl.BlockSpec((8, 128), lambda i: (0, 0)),
        ],
        out_shape=[
            jax.ShapeDtypeStruct((N, 128), jnp.float32),
            jax.ShapeDtypeStruct((8, 128), jnp.float32),
        ],
    )(f, f1, z2, mx2, mu, iv, w3t)


def _heada_body(g3_ref, xm_ref, m3_ref, i3_ref, mp_ref, ip_ref,
                w0_ref, b0_ref, h0_ref, st_ref):
    i = pl.program_id(0)
    fgcn = _lrelu((g3_ref[...] - m3_ref[...]) * i3_ref[...])
    fppf = jnp.maximum((xm_ref[...] - mp_ref[...]) * ip_ref[...], 0.0)
    h = jnp.concatenate([fppf, fgcn], axis=1)
    h0 = jnp.dot(h, w0_ref[...], preferred_element_type=jnp.float32) + b0_ref[...]
    h0_ref[...] = h0

    @pl.when(i == 0)
    def _():
        st_ref[...] = jnp.zeros_like(st_ref)

    st_ref[0:1, :] += jnp.sum(h0, axis=0)[None, :]
    st_ref[1:2, :] += jnp.sum(h0 * h0, axis=0)[None, :]


def _heada(g3, xmt, m3, i3, mp, ip, w0t, b0):
    return pl.pallas_call(
        _heada_body,
        grid=(N // 512,),
        in_specs=[
            pl.BlockSpec((512, 128), lambda i: (i, 0)),
            pl.BlockSpec((512, 64), lambda i: (i, 0)),
            pl.BlockSpec((1, 128), lambda i: (0, 0)),
            pl.BlockSpec((1, 128), lambda i: (0, 0)),
            pl.BlockSpec((1, 64), lambda i: (0, 0)),
            pl.BlockSpec((1, 64), lambda i: (0, 0)),
            pl.BlockSpec((192, 192), lambda i: (0, 0)),
            pl.BlockSpec((1, 192), lambda i: (0, 0)),
        ],
        out_specs=[
            pl.BlockSpec((512, 192), lambda i: (i, 0)),
            pl.BlockSpec((8, 192), lambda i: (0, 0)),
        ],
        out_shape=[
            jax.ShapeDtypeStruct((N, 192), jnp.float32),
            jax.ShapeDtypeStruct((8, 192), jnp.float32),
        ],
    )(g3, xmt, m3, i3, mp, ip, w0t, b0)


def _headb_body(h0_ref, m_ref, iv_ref, w1_ref, b1_ref, h1_ref, st_ref):
    i = pl.program_id(0)
    a0 = _lrelu((h0_ref[...] - m_ref[...]) * iv_ref[...])
    h1 = jnp.dot(a0, w1_ref[...], preferred_element_type=jnp.float32) + b1_ref[...]
    h1_ref[...] = h1

    @pl.when(i == 0)
    def _():
        st_ref[...] = jnp.zeros_like(st_ref)

    st_ref[0:1, :] += jnp.sum(h1, axis=0)[None, :]
    st_ref[1:2, :] += jnp.sum(h1 * h1, axis=0)[None, :]


def _headb(h0, m, iv, w1t, b1):
    return pl.pallas_call(
        _headb_body,
        grid=(N // 512,),
        in_specs=[
            pl.BlockSpec((512, 192), lambda i: (i, 0)),
            pl.BlockSpec((1, 192), lambda i: (0, 0)),
            pl.BlockSpec((1, 192), lambda i: (0, 0)),
            pl.BlockSpec((192, 128), lambda i: (0, 0)),
            pl.BlockSpec((1, 128), lambda i: (0, 0)),
        ],
        out_specs=[
            pl.BlockSpec((512, 128), lambda i: (i, 0)),
            pl.BlockSpec((8, 128), lambda i: (0, 0)),
        ],
        out_shape=[
            jax.ShapeDtypeStruct((N, 128), jnp.float32),
            jax.ShapeDtypeStruct((8, 128), jnp.float32),
        ],
    )(h0, m, iv, w1t, b1)


def _headc_body(h1_ref, m_ref, iv_ref, o_ref):
    o_ref[...] = _lrelu((h1_ref[...] - m_ref[...]) * iv_ref[...])


def _headc(h1, m, iv):
    return pl.pallas_call(
        _headc_body,
        grid=(N // 512,),
        in_specs=[
            pl.BlockSpec((512, 128), lambda i: (i, 0)),
            pl.BlockSpec((1, 128), lambda i: (0, 0)),
            pl.BlockSpec((1, 128), lambda i: (0, 0)),
        ],
        out_specs=pl.BlockSpec((512, 128), lambda i: (i, 0)),
        out_shape=jax.ShapeDtypeStruct((N, 128), jnp.float32),
    )(h1, m, iv)


def _mi(st, count, C, rowvec):
    s = st[0, :C]
    sq = st[1, :C]
    m = s / count
    v = sq / count - m * m
    iv = lax.rsqrt(v + EPS)
    if rowvec:
        return m[None, :], iv[None, :]
    return m[:, None], iv[:, None]


def kernel(coords, feats, normals, gcn_w1, gcn_w2, gcn_w3, ppf_w0, ppf_w1,
           ppf_w2, fused_w0, fused_b0, fused_w1, fused_b1):
    P = coords[0].T
    F = feats[0].T
    Nm = normals[0].T

    ppad = jnp.pad(P, ((0, 0), (0, 5)))
    ppad_t = ppad.T
    sel = _select(ppad, ppad_t)
    knn_flat = sel[:, 1:K + 1].reshape(NK)
    ball_flat = sel[:, 64:64 + K].T.reshape(NK)

    tbl8 = jnp.pad(jnp.concatenate([P, Nm], axis=1), ((0, 0), (0, 122)))
    gplanes = _sc_ppf_gather(tbl8, ball_flat)
    pt4 = jnp.pad(P.T, ((0, 1), (0, 0)))
    nt4 = jnp.pad(Nm.T, ((0, 1), (0, 0)))
    w0p = jnp.pad(ppf_w0, ((0, 0), (0, 6)))
    f10, st0 = _ppf_ang(gplanes, pt4, nt4, w0p)
    m0, i0 = _mi(st0, NK, 64, rowvec=False)
    st1 = _ppf_mid(f10, w0p, ppf_w1, m0, i0)
    m1, i1 = _mi(st1, NK, 128, rowvec=False)
    xmax, st2 = _ppf_last(f10, w0p, ppf_w1, ppf_w2, m0, i0, m1, i1)
    mp, ip = _mi(st2, NK, 64, rowvec=True)

    wc1, wn1 = gcn_w1[:, :128], gcn_w1[:, 128:]
    y1, z1 = _mm2(F, wn1.T, (wc1 - wn1).T)
    mx1, parts1 = _sc_ec_reduce(y1, knn_flat, z1)
    mu1, iv1 = _mi(jnp.sum(parts1, axis=0), NK, 128, rowvec=True)

    wc2, wn2 = gcn_w2[:, :128], gcn_w2[:, 128:]
    f1, y2, z2 = _gcnb(z1, mx1, mu1, iv1, wn2.T, (wc2 - wn2).T)
    mx2a, parts2a = _sc_ec_reduce(y2[:, :128], knn_flat, z2[:, :128])
    mx2b, parts2b = _sc_ec_reduce(y2[:, 128:], knn_flat, z2[:, 128:],
                                  use_spmem=False)
    mx2 = jnp.concatenate([mx2a, mx2b], axis=1)
    st2 = jnp.concatenate(
        [jnp.sum(parts2a, axis=0), jnp.sum(parts2b, axis=0)], axis=1)
    mu2, iv2 = _mi(st2, NK, 256, rowvec=True)

    g3, st3 = _gcnc(F, f1, z2, mx2, mu2, iv2, gcn_w3.T)
    m3, i3 = _mi(st3, N, 128, rowvec=True)

    h0, stH0 = _heada(g3, xmax.T, m3, i3, mp, ip, fused_w0.T,
                      fused_b0[None, :])
    mh0, ih0 = _mi(stH0, N, 192, rowvec=True)
    h1, stH1 = _headb(h0, mh0, ih0, fused_w1.T, fused_b1[None, :])
    mh1, ih1 = _mi(stH1, N, 128, rowvec=True)
    out = _headc(h1, mh1, ih1)
    return out.T[None]


# speedup vs baseline: 1.0002x; 1.0002x over previous
import functools

import jax
import jax.numpy as jnp
from jax import lax
from jax.experimental import pallas as pl
from jax.experimental.pallas import tpu as pltpu
from jax.experimental.pallas import tpu_sc as plsc

EPS = 1e-5
K = 32
R2 = 0.3 * 0.3
N = 4096
NK = N * K
NW = 32


def _sel_body(pb_ref, pt_ref, out_ref, d_ref):
    pb = pb_ref[...]
    pt = pt_ref[...]
    srow = jnp.sum(pb * pb, axis=1, keepdims=True)
    scol = jnp.sum(pt * pt, axis=0, keepdims=True)
    D = srow + scol - 2.0 * jnp.dot(pb, pt, preferred_element_type=jnp.float32)

    iot = lax.broadcasted_iota(jnp.int32, (128, N), 1)
    col = lax.broadcasted_iota(jnp.int32, (128, 128), 1)

    mask = D <= R2
    cnt = mask.astype(jnp.int32)
    sh = 1
    while sh < N:
        cnt = cnt + jnp.concatenate(
            [jnp.zeros((128, sh), jnp.int32), cnt[:, : N - sh]], axis=1)
        sh *= 2
    ball0 = jnp.min(jnp.where(mask & (cnt == 1), iot, N), axis=1)
    res = jnp.where(col == 64, ball0[:, None], jnp.zeros((128, 128), jnp.int32))

    def ball_step(s, res):
        cand = jnp.where(mask & (cnt == s + 1), iot, N)
        idx = jnp.min(cand, axis=1)
        idx = jnp.where(idx == N, ball0, idx)
        return jnp.where(col == 64 + s, idx[:, None], res)

    res = lax.fori_loop(1, K, ball_step, res)

    d_ref[...] = D

    def topk_step(t, res):
        Dw = d_ref[...]
        m = jnp.min(Dw, axis=1, keepdims=True)
        arg = jnp.min(jnp.where(Dw == m, iot, N), axis=1)
        d_ref[...] = jnp.where(iot == arg[:, None], jnp.float32(jnp.inf), Dw)
        return jnp.where(col == t, arg[:, None], res)

    res = lax.fori_loop(0, K + 1, topk_step, res)
    out_ref[...] = res


def _select(ppad, ppad_t):
    return pl.pallas_call(
        _sel_body,
        grid=(N // 128,),
        in_specs=[
            pl.BlockSpec((128, 8), lambda i: (i, 0)),
            pl.BlockSpec((8, N), lambda i: (0, 0)),
        ],
        out_specs=pl.BlockSpec((128, 128), lambda i: (i, 0)),
        out_shape=jax.ShapeDtypeStruct((N, 128), jnp.int32),
        scratch_shapes=[pltpu.VMEM((128, N), jnp.float32)],
    )(ppad, ppad_t)


def _sc_ec_reduce(y, idxflat, z, use_spmem=True):
    C = y.shape[1]
    nodes_pw = N // NW
    rows_pw = nodes_pw * K
    CH = 8 if C <= 128 else 4
    chunk = CH * K
    nch = nodes_pw // CH
    NCH = C // 16
    mesh = plsc.VectorSubcoreMesh(core_axis_name="c", subcore_axis_name="s")

    @functools.partial(
        pl.kernel,
        mesh=mesh,
        out_type=[
            jax.ShapeDtypeStruct((N, C), jnp.float32),
            jax.ShapeDtypeStruct((NW, 8, C), jnp.float32),
        ],
        scratch_types=[
            pltpu.VMEM((rows_pw,), jnp.int32),
            pltpu.VMEM((2, chunk, C), jnp.float32),
            pltpu.VMEM((nodes_pw, C), jnp.float32),
            pltpu.VMEM((CH, C), jnp.float32),
            pltpu.VMEM((8, C), jnp.float32),
            (pltpu.VMEM_SHARED((N, C), jnp.float32) if use_spmem
             else pltpu.VMEM((8,), jnp.float32)),
            pltpu.SemaphoreType.DMA,
        ],
    )
    def k(y_hbm, idx_hbm, z_hbm, m_hbm, part_hbm, idx_v, rows_v, m_v, z_v,
          p_v, ysh, gsem):
        wid = lax.axis_index("s") * 2 + lax.axis_index("c")
        base_row = wid * rows_pw
        base_node = wid * nodes_pw
        if use_spmem:
            ytab = ysh
            sid = lax.axis_index("s")
            stage = N // 16
            so = pl.multiple_of(sid * stage, stage)
            pltpu.sync_copy(y_hbm.at[pl.ds(so, stage)],
                            ysh.at[pl.ds(so, stage)])
        else:
            ytab = y_hbm
        pltpu.sync_copy(idx_hbm.at[pl.ds(base_row, rows_pw)], idx_v)
        if use_spmem:
            plsc.subcore_barrier()
        for cc in range(NCH):
            zv = jnp.zeros((16,), jnp.float32)
            p_v[0, pl.ds(cc * 16, 16)] = zv
            p_v[1, pl.ds(cc * 16, 16)] = zv

        def gstart(j, b):
            off = pl.multiple_of(j * chunk, chunk)
            pltpu.async_copy(
                ytab.at[idx_v.at[pl.ds(off, chunk)]], rows_v.at[b], gsem)

        def gwait():
            pltpu.make_async_copy(
                ytab.at[idx_v.at[pl.ds(0, chunk)]], rows_v.at[0], gsem
            ).wait()

        gstart(0, 0)

        def outer(jh, _):
            for b in range(2):
                j = jh * 2 + b

                @pl.when(j + 1 < nch)
                def _():
                    gstart(j + 1, 1 - b)

                pltpu.sync_copy(
                    z_hbm.at[pl.ds(base_node + j * CH, CH)], z_v)
                gwait()
                for i in range(CH):
                    r0 = i * K
                    for cc in range(NCH):
                        c0 = cc * 16
                        ga0 = rows_v[b, r0, pl.ds(c0, 16)]
                        gb0 = rows_v[b, r0 + K // 2, pl.ds(c0, 16)]

                        def kstep(kk, acc):
                            ga = rows_v[b, r0 + kk, pl.ds(c0, 16)]
                            gb = rows_v[b, r0 + K // 2 + kk, pl.ds(c0, 16)]
                            return (jnp.maximum(acc[0], ga),
                                    jnp.maximum(acc[1], gb),
                                    acc[2] + (ga + gb),
                                    acc[3] + (ga * ga + gb * gb))

                        mxa, mxb, s1, s2 = lax.fori_loop(
                            1, K // 2, kstep,
                            (ga0, gb0, ga0 + gb0, ga0 * ga0 + gb0 * gb0))
                        mx = jnp.maximum(mxa, mxb)
                        m_v[j * CH + i, pl.ds(c0, 16)] = mx
                        zr = z_v[i, pl.ds(c0, 16)]
                        p_v[0, pl.ds(c0, 16)] += K * zr + s1
                        p_v[1, pl.ds(c0, 16)] += (
                            K * zr * zr + 2.0 * zr * s1 + s2)
            return 0

        lax.fori_loop(0, nch // 2, outer, 0)
        pltpu.sync_copy(m_v, m_hbm.at[pl.ds(base_node, nodes_pw)])
        pltpu.sync_copy(p_v, part_hbm.at[wid])

    return k(y, idxflat, z)


def _sc_ppf_gather(tbl8, ballflat):
    rows_pw = NK // NW
    mesh = plsc.VectorSubcoreMesh(core_axis_name="c", subcore_axis_name="s")

    chunk = 256
    nch = rows_pw // chunk

    @functools.partial(
        pl.kernel,
        mesh=mesh,
        out_type=jax.ShapeDtypeStruct((NK, 128), jnp.float32),
        scratch_types=[
            pltpu.VMEM((rows_pw,), jnp.int32),
            pltpu.VMEM((2, chunk, 128), jnp.float32),
            pltpu.VMEM_SHARED((N, 128), jnp.float32),
            pltpu.SemaphoreType.DMA,
        ],
    )
    def k(tbl_hbm, idx_hbm, out_hbm, idx_v, rows_v, sh, sem):
        sid = lax.axis_index("s")
        wid = sid * 2 + lax.axis_index("c")
        base = wid * rows_pw
        stage = N // 16
        so = pl.multiple_of(sid * stage, stage)
        pltpu.sync_copy(tbl_hbm.at[pl.ds(so, stage)], sh.at[pl.ds(so, stage)])
        pltpu.sync_copy(idx_hbm.at[pl.ds(base, rows_pw)], idx_v)
        plsc.subcore_barrier()

        def gstart(j, b):
            off = pl.multiple_of(j * chunk, chunk)
            pltpu.async_copy(
                sh.at[idx_v.at[pl.ds(off, chunk)]], rows_v.at[b], sem)

        def gwait():
            pltpu.make_async_copy(
                sh.at[idx_v.at[pl.ds(0, chunk)]], rows_v.at[0],
                sem).wait()

        gstart(0, 0)
        for j in range(nch):
            if j + 1 < nch:
                gstart(j + 1, (j + 1) % 2)
            gwait()
            pltpu.sync_copy(rows_v.at[j % 2],
                            out_hbm.at[pl.ds(base + j * chunk, chunk)])

    return k(tbl8, ballflat)


def _ppf_ang_body(gp_ref, pt_ref, nt_ref, w0_ref, f10_ref, st_ref):
    kstep = pl.program_id(0)
    g = gp_ref[...]
    gp = jnp.transpose(g)
    px, py, pz = pt_ref[0:1, :], pt_ref[1:2, :], pt_ref[2:3, :]
    nix, niy, niz = nt_ref[0:1, :], nt_ref[1:2, :], nt_ref[2:3, :]
    gx = gp[0:1, :] - px
    gy = gp[1:2, :] - py
    gz = gp[2:3, :] - pz
    njx, njy, njz = gp[3:4, :], gp[4:5, :], gp[5:6, :]

    def ang(ax, ay, az, bx, by, bz):
        cx = ay * bz - az * by
        cy = az * bx - ax * bz
        cz = ax * by - ay * bx
        yv = jnp.sqrt(cx * cx + cy * cy + cz * cz + 1e-12)
        xv = ax * bx + ay * by + az * bz
        return jnp.arctan2(yv, xv)

    nr_d = ang(nix, niy, niz, gx, gy, gz)
    ni_d = ang(njx, njy, njz, gx, gy, gz)
    nr_ni = ang(nix, niy, niz, njx, njy, njz)
    dn = jnp.sqrt(gx * gx + gy * gy + gz * gz + 1e-12)
    zr = jnp.zeros((6, N), jnp.float32)
    f10 = jnp.concatenate(
        [px, py, pz, gx, gy, gz, nr_d, ni_d, nr_ni, dn, zr], axis=0)
    f10_ref[...] = f10

    x0 = jnp.dot(w0_ref[...], f10, preferred_element_type=jnp.float32)

    @pl.when(kstep == 0)
    def _():
        st_ref[...] = jnp.zeros_like(st_ref)

    st_ref[0:1, 0:64] += jnp.sum(x0, axis=1)[None, :]
    st_ref[1:2, 0:64] += jnp.sum(x0 * x0, axis=1)[None, :]


def _ppf_ang(gplanes, pt, nt, w0p):
    return pl.pallas_call(
        _ppf_ang_body,
        grid=(K,),
        in_specs=[
            pl.BlockSpec((N, 128), lambda k: (k, 0)),
            pl.BlockSpec((4, N), lambda k: (0, 0)),
            pl.BlockSpec((4, N), lambda k: (0, 0)),
            pl.BlockSpec((64, 16), lambda k: (0, 0)),
        ],
        out_specs=[
            pl.BlockSpec((16, N), lambda k: (0, k)),
            pl.BlockSpec((8, 128), lambda k: (0, 0)),
        ],
        out_shape=[
            jax.ShapeDtypeStruct((16, NK), jnp.float32),
            jax.ShapeDtypeStruct((8, 128), jnp.float32),
        ],
    )(gplanes, pt, nt, w0p)


def _ppf_mid_body(f10_ref, w0_ref, w1_ref, m0_ref, i0_ref, st_ref):
    kstep = pl.program_id(0)
    x0 = jnp.dot(w0_ref[...], f10_ref[...], preferred_element_type=jnp.float32)
    h0 = (x0 - m0_ref[...]) * i0_ref[...]
    h0 = jnp.maximum(h0, 0.0)
    x1 = jnp.dot(w1_ref[...], h0, preferred_element_type=jnp.float32)

    @pl.when(kstep == 0)
    def _():
        st_ref[...] = jnp.zeros_like(st_ref)

    st_ref[0:1, :] += jnp.sum(x1, axis=1)[None, :]
    st_ref[1:2, :] += jnp.sum(x1 * x1, axis=1)[None, :]


def _ppf_mid(f10, w0p, w1, m0, i0):
    return pl.pallas_call(
        _ppf_mid_body,
        grid=(K,),
        in_specs=[
            pl.BlockSpec((16, N), lambda k: (0, k)),
            pl.BlockSpec((64, 16), lambda k: (0, 0)),
            pl.BlockSpec((128, 64), lambda k: (0, 0)),
            pl.BlockSpec((64, 1), lambda k: (0, 0)),
            pl.BlockSpec((64, 1), lambda k: (0, 0)),
        ],
        out_specs=pl.BlockSpec((8, 128), lambda k: (0, 0)),
        out_shape=jax.ShapeDtypeStruct((8, 128), jnp.float32),
    )(f10, w0p, w1, m0, i0)


def _ppf_last_body(f10_ref, w0_ref, w1_ref, w2_ref, m0_ref, i0_ref,
                   m1_ref, i1_ref, xmax_ref, st_ref):
    kstep = pl.program_id(0)
    x0 = jnp.dot(w0_ref[...], f10_ref[...], preferred_element_type=jnp.float32)
    h0 = jnp.maximum((x0 - m0_ref[...]) * i0_ref[...], 0.0)
    x1 = jnp.dot(w1_ref[...], h0, preferred_element_type=jnp.float32)
    h1 = jnp.maximum((x1 - m1_ref[...]) * i1_ref[...], 0.0)
    x2 = jnp.dot(w2_ref[...], h1, preferred_element_type=jnp.float32)

    @pl.when(kstep == 0)
    def _():
        xmax_ref[...] = x2
        st_ref[...] = jnp.zeros_like(st_ref)

    @pl.when(kstep > 0)
    def _():
        xmax_ref[...] = jnp.maximum(xmax_ref[...], x2)

    st_ref[0:1, 0:64] += jnp.sum(x2, axis=1)[None, :]
    st_ref[1:2, 0:64] += jnp.sum(x2 * x2, axis=1)[None, :]


def _ppf_last(f10, w0p, w1, w2, m0, i0, m1, i1):
    return pl.pallas_call(
        _ppf_last_body,
        grid=(K,),
        in_specs=[
            pl.BlockSpec((16, N), lambda k: (0, k)),
            pl.BlockSpec((64, 16), lambda k: (0, 0)),
            pl.BlockSpec((128, 64), lambda k: (0, 0)),
            pl.BlockSpec((64, 128), lambda k: (0, 0)),
            pl.BlockSpec((64, 1), lambda k: (0, 0)),
            pl.BlockSpec((64, 1), lambda k: (0, 0)),
            pl.BlockSpec((128, 1), lambda k: (0, 0)),
            pl.BlockSpec((128, 1), lambda k: (0, 0)),
        ],
        out_specs=[
            pl.BlockSpec((64, N), lambda k: (0, 0)),
            pl.BlockSpec((8, 128), lambda k: (0, 0)),
        ],
        out_shape=[
            jax.ShapeDtypeStruct((64, N), jnp.float32),
            jax.ShapeDtypeStruct((8, 128), jnp.float32),
        ],
    )(f10, w0p, w1, w2, m0, i0, m1, i1)


def _mm2_body(f_ref, wa_ref, wb_ref, ya_ref, yb_ref):
    f = f_ref[...]
    ya_ref[...] = jnp.dot(f, wa_ref[...], preferred_element_type=jnp.float32)
    yb_ref[...] = jnp.dot(f, wb_ref[...], preferred_element_type=jnp.float32)


def _mm2(f, wa, wb):
    Cin = f.shape[1]
    Ca, Cb = wa.shape[1], wb.shape[1]
    return pl.pallas_call(
        _mm2_body,
        grid=(N // 512,),
        in_specs=[
            pl.BlockSpec((512, Cin), lambda i: (i, 0)),
            pl.BlockSpec((Cin, Ca), lambda i: (0, 0)),
            pl.BlockSpec((Cin, Cb), lambda i: (0, 0)),
        ],
        out_specs=[
            pl.BlockSpec((512, Ca), lambda i: (i, 0)),
            pl.BlockSpec((512, Cb), lambda i: (i, 0)),
        ],
        out_shape=[
            jax.ShapeDtypeStruct((N, Ca), jnp.float32),
            jax.ShapeDtypeStruct((N, Cb), jnp.float32),
        ],
    )(f, wa, wb)


def _lrelu(x):
    return jnp.where(x >= 0, x, 0.2 * x)


def _gcnb_body(z_ref, mx_ref, mu_ref, iv_ref, wa_ref, wb_ref,
               f1_ref, ya_ref, yb_ref):
    f1 = _lrelu((z_ref[...] + mx_ref[...] - mu_ref[...]) * iv_ref[...])
    f1_ref[...] = f1
    ya_ref[...] = jnp.dot(f1, wa_ref[...], preferred_element_type=jnp.float32)
    yb_ref[...] = jnp.dot(f1, wb_ref[...], preferred_element_type=jnp.float32)


def _gcnb(z, mx, mu, iv, wa, wb):
    Cin = z.shape[1]
    Ca, Cb = wa.shape[1], wb.shape[1]
    return pl.pallas_call(
        _gcnb_body,
        grid=(N // 512,),
        in_specs=[
            pl.BlockSpec((512, Cin), lambda i: (i, 0)),
            pl.BlockSpec((512, Cin), lambda i: (i, 0)),
            pl.BlockSpec((1, Cin), lambda i: (0, 0)),
            pl.BlockSpec((1, Cin), lambda i: (0, 0)),
            pl.BlockSpec((Cin, Ca), lambda i: (0, 0)),
            pl.BlockSpec((Cin, Cb), lambda i: (0, 0)),
        ],
        out_specs=[
            pl.BlockSpec((512, Cin), lambda i: (i, 0)),
            pl.BlockSpec((512, Ca), lambda i: (i, 0)),
            pl.BlockSpec((512, Cb), lambda i: (i, 0)),
        ],
        out_shape=[
            jax.ShapeDtypeStruct((N, Cin), jnp.float32),
            jax.ShapeDtypeStruct((N, Ca), jnp.float32),
            jax.ShapeDtypeStruct((N, Cb), jnp.float32),
        ],
    )(z, mx, mu, iv, wa, wb)


def _gcnc_body(f_ref, f1_ref, z2_ref, mx2_ref, mu_ref, iv_ref, w3_ref,
               g3_ref, st_ref):
    i = pl.program_id(0)
    f2 = _lrelu((z2_ref[...] + mx2_ref[...] - mu_ref[...]) * iv_ref[...])
    f3 = jnp.concatenate([f_ref[...], f1_ref[...], f2], axis=1)
    g3 = jnp.dot(f3, w3_ref[...], preferred_element_type=jnp.float32)
    g3_ref[...] = g3

    @pl.when(i == 0)
    def _():
        st_ref[...] = jnp.zeros_like(st_ref)

    st_ref[0:1, :] += jnp.sum(g3, axis=0)[None, :]
    st_ref[1:2, :] += jnp.sum(g3 * g3, axis=0)[None, :]


def _gcnc(f, f1, z2, mx2, mu, iv, w3t):
    return pl.pallas_call(
        _gcnc_body,
        grid=(N // 512,),
        in_specs=[
            pl.BlockSpec((512, 128), lambda i: (i, 0)),
            pl.BlockSpec((512, 128), lambda i: (i, 0)),
            pl.BlockSpec((512, 256), lambda i: (i, 0)),
            pl.BlockSpec((512, 256), lambda i: (i, 0)),
            pl.BlockSpec((1, 256), lambda i: (0, 0)),
            pl.BlockSpec((1, 256), lambda i: (0, 0)),
            pl.BlockSpec((512, 128), lambda i: (0, 0)),
        ],
        out_specs=[
            pl.BlockSpec((512, 128), lambda i: (i, 0)),
            pl.BlockSpec((8, 128), lambda i: (0, 0)),
        ],
        out_shape=[
            jax.ShapeDtypeStruct((N, 128), jnp.float32),
            jax.ShapeDtypeStruct((8, 128), jnp.float32),
        ],
    )(f, f1, z2, mx2, mu, iv, w3t)


def _heada_body(g3_ref, xm_ref, m3_ref, i3_ref, mp_ref, ip_ref,
                w0_ref, b0_ref, h0_ref, st_ref):
    i = pl.program_id(0)
    fgcn = _lrelu((g3_ref[...] - m3_ref[...]) * i3_ref[...])
    fppf = jnp.maximum((xm_ref[...] - mp_ref[...]) * ip_ref[...], 0.0)
    h = jnp.concatenate([fppf, fgcn], axis=1)
    h0 = jnp.dot(h, w0_ref[...], preferred_element_type=jnp.float32) + b0_ref[...]
    h0_ref[...] = h0

    @pl.when(i == 0)
    def _():
        st_ref[...] = jnp.zeros_like(st_ref)

    st_ref[0:1, :] += jnp.sum(h0, axis=0)[None, :]
    st_ref[1:2, :] += jnp.sum(h0 * h0, axis=0)[None, :]


def _heada(g3, xmt, m3, i3, mp, ip, w0t, b0):
    return pl.pallas_call(
        _heada_body,
        grid=(N // 512,),
        in_specs=[
            pl.BlockSpec((512, 128), lambda i: (i, 0)),
            pl.BlockSpec((512, 64), lambda i: (i, 0)),
            pl.BlockSpec((1, 128), lambda i: (0, 0)),
            pl.BlockSpec((1, 128), lambda i: (0, 0)),
            pl.BlockSpec((1, 64), lambda i: (0, 0)),
            pl.BlockSpec((1, 64), lambda i: (0, 0)),
            pl.BlockSpec((192, 192), lambda i: (0, 0)),
            pl.BlockSpec((1, 192), lambda i: (0, 0)),
        ],
        out_specs=[
            pl.BlockSpec((512, 192), lambda i: (i, 0)),
            pl.BlockSpec((8, 192), lambda i: (0, 0)),
        ],
        out_shape=[
            jax.ShapeDtypeStruct((N, 192), jnp.float32),
            jax.ShapeDtypeStruct((8, 192), jnp.float32),
        ],
    )(g3, xmt, m3, i3, mp, ip, w0t, b0)


def _headb_body(h0_ref, m_ref, iv_ref, w1_ref, b1_ref, h1_ref, st_ref):
    i = pl.program_id(0)
    a0 = _lrelu((h0_ref[...] - m_ref[...]) * iv_ref[...])
    h1 = jnp.dot(a0, w1_ref[...], preferred_element_type=jnp.float32) + b1_ref[...]
    h1_ref[...] = h1

    @pl.when(i == 0)
    def _():
        st_ref[...] = jnp.zeros_like(st_ref)

    st_ref[0:1, :] += jnp.sum(h1, axis=0)[None, :]
    st_ref[1:2, :] += jnp.sum(h1 * h1, axis=0)[None, :]


def _headb(h0, m, iv, w1t, b1):
    return pl.pallas_call(
        _headb_body,
        grid=(N // 512,),
        in_specs=[
            pl.BlockSpec((512, 192), lambda i: (i, 0)),
            pl.BlockSpec((1, 192), lambda i: (0, 0)),
            pl.BlockSpec((1, 192), lambda i: (0, 0)),
            pl.BlockSpec((192, 128), lambda i: (0, 0)),
            pl.BlockSpec((1, 128), lambda i: (0, 0)),
        ],
        out_specs=[
            pl.BlockSpec((512, 128), lambda i: (i, 0)),
            pl.BlockSpec((8, 128), lambda i: (0, 0)),
        ],
        out_shape=[
            jax.ShapeDtypeStruct((N, 128), jnp.float32),
            jax.ShapeDtypeStruct((8, 128), jnp.float32),
        ],
    )(h0, m, iv, w1t, b1)


def _headc_body(h1_ref, m_ref, iv_ref, o_ref):
    o_ref[...] = _lrelu((h1_ref[...] - m_ref[...]) * iv_ref[...])


def _headc(h1, m, iv):
    return pl.pallas_call(
        _headc_body,
        grid=(N // 512,),
        in_specs=[
            pl.BlockSpec((512, 128), lambda i: (i, 0)),
            pl.BlockSpec((1, 128), lambda i: (0, 0)),
            pl.BlockSpec((1, 128), lambda i: (0, 0)),
        ],
        out_specs=pl.BlockSpec((512, 128), lambda i: (i, 0)),
        out_shape=jax.ShapeDtypeStruct((N, 128), jnp.float32),
    )(h1, m, iv)


def _mi(st, count, C, rowvec):
    s = st[0, :C]
    sq = st[1, :C]
    m = s / count
    v = sq / count - m * m
    iv = lax.rsqrt(v + EPS)
    if rowvec:
        return m[None, :], iv[None, :]
    return m[:, None], iv[:, None]


def kernel(coords, feats, normals, gcn_w1, gcn_w2, gcn_w3, ppf_w0, ppf_w1,
           ppf_w2, fused_w0, fused_b0, fused_w1, fused_b1):
    P = coords[0].T
    F = feats[0].T
    Nm = normals[0].T

    ppad = jnp.pad(P, ((0, 0), (0, 5)))
    ppad_t = ppad.T
    sel = _select(ppad, ppad_t)
    knn_flat = sel[:, 1:K + 1].reshape(NK)
    ball_flat = sel[:, 64:64 + K].T.reshape(NK)

    tbl8 = jnp.pad(jnp.concatenate([P, Nm], axis=1), ((0, 0), (0, 122)))
    gplanes = _sc_ppf_gather(tbl8, ball_flat)
    pt4 = jnp.pad(P.T, ((0, 1), (0, 0)))
    nt4 = jnp.pad(Nm.T, ((0, 1), (0, 0)))
    w0p = jnp.pad(ppf_w0, ((0, 0), (0, 6)))
    f10, st0 = _ppf_ang(gplanes, pt4, nt4, w0p)
    m0, i0 = _mi(st0, NK, 64, rowvec=False)
    st1 = _ppf_mid(f10, w0p, ppf_w1, m0, i0)
    m1, i1 = _mi(st1, NK, 128, rowvec=False)
    xmax, st2 = _ppf_last(f10, w0p, ppf_w1, ppf_w2, m0, i0, m1, i1)
    mp, ip = _mi(st2, NK, 64, rowvec=True)

    wc1, wn1 = gcn_w1[:, :128], gcn_w1[:, 128:]
    y1, z1 = _mm2(F, wn1.T, (wc1 - wn1).T)
    mx1, parts1 = _sc_ec_reduce(y1, knn_flat, z1)
    mu1, iv1 = _mi(jnp.sum(parts1, axis=0), NK, 128, rowvec=True)

    wc2, wn2 = gcn_w2[:, :128], gcn_w2[:, 128:]
    f1, y2, z2 = _gcnb(z1, mx1, mu1, iv1, wn2.T, (wc2 - wn2).T)
    mx2a, parts2a = _sc_ec_reduce(y2[:, :128], knn_flat, z2[:, :128])
    mx2b, parts2b = _sc_ec_reduce(y2[:, 128:], knn_flat, z2[:, 128:],
                                  use_spmem=False)
    mx2 = jnp.concatenate([mx2a, mx2b], axis=1)
    st2 = jnp.concatenate(
        [jnp.sum(parts2a, axis=0), jnp.sum(parts2b, axis=0)], axis=1)
    mu2, iv2 = _mi(st2, NK, 256, rowvec=True)

    g3, st3 = _gcnc(F, f1, z2, mx2, mu2, iv2, gcn_w3.T)
    m3, i3 = _mi(st3, N, 128, rowvec=True)

    h0, stH0 = _heada(g3, xmax.T, m3, i3, mp, ip, fused_w0.T,
                      fused_b0[None, :])
    mh0, ih0 = _mi(stH0, N, 192, rowvec=True)
    h1, stH1 = _headb(h0, mh0, ih0, fused_w1.T, fused_b1[None, :])
    mh1, ih1 = _mi(stH1, N, 128, rowvec=True)
    out = _headc(h1, mh1, ih1)
    return out.T[None]


# speedup vs baseline: 1.0915x; 1.0913x over previous
import functools

import jax
import jax.numpy as jnp
from jax import lax
from jax.experimental import pallas as pl
from jax.experimental.pallas import tpu as pltpu
from jax.experimental.pallas import tpu_sc as plsc

EPS = 1e-5
K = 32
R2 = 0.3 * 0.3
N = 4096
NK = N * K
NW = 32


def _sel_body(pb_ref, pt_ref, out_ref, d_ref):
    R = 256
    pb = pb_ref[...]
    pt = pt_ref[...]
    srow = jnp.sum(pb * pb, axis=1, keepdims=True)
    scol = jnp.sum(pt * pt, axis=0, keepdims=True)
    D = srow + scol - 2.0 * jnp.dot(pb, pt, preferred_element_type=jnp.float32)

    iot = lax.broadcasted_iota(jnp.int32, (R, N), 1)
    col = lax.broadcasted_iota(jnp.int32, (R, 128), 1)

    mask = D <= R2
    cnt = mask.astype(jnp.int32)
    sh = 1
    while sh < N:
        cnt = cnt + jnp.concatenate(
            [jnp.zeros((R, sh), jnp.int32), cnt[:, : N - sh]], axis=1)
        sh *= 2
    ball0 = jnp.min(jnp.where(mask & (cnt == 1), iot, N), axis=1)
    res = jnp.where(col == 64, ball0[:, None], jnp.zeros((R, 128), jnp.int32))

    def ball_step(s, res):
        cand = jnp.where(mask & (cnt == s + 1), iot, N)
        idx = jnp.min(cand, axis=1)
        idx = jnp.where(idx == N, ball0, idx)
        return jnp.where(col == 64 + s, idx[:, None], res)

    res = lax.fori_loop(1, K, ball_step, res)

    d_ref[...] = D

    def topk_step(t, res):
        Dw = d_ref[...]
        m = jnp.min(Dw, axis=1, keepdims=True)
        arg = jnp.min(jnp.where(Dw == m, iot, N), axis=1)
        d_ref[...] = jnp.where(iot == arg[:, None], jnp.float32(jnp.inf), Dw)
        return jnp.where(col == t, arg[:, None], res)

    res = lax.fori_loop(0, K + 1, topk_step, res)
    out_ref[...] = res


def _select(ppad, ppad_t):
    return pl.pallas_call(
        _sel_body,
        grid=(N // 256,),
        in_specs=[
            pl.BlockSpec((256, 8), lambda i: (i, 0)),
            pl.BlockSpec((8, N), lambda i: (0, 0)),
        ],
        out_specs=pl.BlockSpec((256, 128), lambda i: (i, 0)),
        out_shape=jax.ShapeDtypeStruct((N, 128), jnp.int32),
        scratch_shapes=[pltpu.VMEM((256, N), jnp.float32)],
    )(ppad, ppad_t)


def _sc_ec_reduce(y, idxflat, z, use_spmem=True):
    C = y.shape[1]
    nodes_pw = N // NW
    rows_pw = nodes_pw * K
    CH = 8 if C <= 128 else 4
    chunk = CH * K
    nch = nodes_pw // CH
    NCH = C // 16
    mesh = plsc.VectorSubcoreMesh(core_axis_name="c", subcore_axis_name="s")

    @functools.partial(
        pl.kernel,
        mesh=mesh,
        out_type=[
            jax.ShapeDtypeStruct((N, C), jnp.float32),
            jax.ShapeDtypeStruct((NW, 8, C), jnp.float32),
        ],
        scratch_types=[
            pltpu.VMEM((rows_pw,), jnp.int32),
            pltpu.VMEM((2, chunk, C), jnp.float32),
            pltpu.VMEM((nodes_pw, C), jnp.float32),
            pltpu.VMEM((CH, C), jnp.float32),
            pltpu.VMEM((8, C), jnp.float32),
            (pltpu.VMEM_SHARED((N, C), jnp.float32) if use_spmem
             else pltpu.VMEM((8,), jnp.float32)),
            pltpu.SemaphoreType.DMA,
        ],
    )
    def k(y_hbm, idx_hbm, z_hbm, m_hbm, part_hbm, idx_v, rows_v, m_v, z_v,
          p_v, ysh, gsem):
        wid = lax.axis_index("s") * 2 + lax.axis_index("c")
        base_row = wid * rows_pw
        base_node = wid * nodes_pw
        if use_spmem:
            ytab = ysh
            sid = lax.axis_index("s")
            stage = N // 16
            so = pl.multiple_of(sid * stage, stage)
            pltpu.sync_copy(y_hbm.at[pl.ds(so, stage)],
                            ysh.at[pl.ds(so, stage)])
        else:
            ytab = y_hbm
        pltpu.sync_copy(idx_hbm.at[pl.ds(base_row, rows_pw)], idx_v)
        if use_spmem:
            plsc.subcore_barrier()
        for cc in range(NCH):
            zv = jnp.zeros((16,), jnp.float32)
            p_v[0, pl.ds(cc * 16, 16)] = zv
            p_v[1, pl.ds(cc * 16, 16)] = zv

        def gstart(j, b):
            off = pl.multiple_of(j * chunk, chunk)
            pltpu.async_copy(
                ytab.at[idx_v.at[pl.ds(off, chunk)]], rows_v.at[b], gsem)

        def gwait():
            pltpu.make_async_copy(
                ytab.at[idx_v.at[pl.ds(0, chunk)]], rows_v.at[0], gsem
            ).wait()

        gstart(0, 0)

        def outer(jh, _):
            for b in range(2):
                j = jh * 2 + b

                @pl.when(j + 1 < nch)
                def _():
                    gstart(j + 1, 1 - b)

                pltpu.sync_copy(
                    z_hbm.at[pl.ds(base_node + j * CH, CH)], z_v)
                gwait()
                for i in range(CH):
                    r0 = i * K
                    for cc in range(NCH):
                        c0 = cc * 16
                        ga0 = rows_v[b, r0, pl.ds(c0, 16)]
                        gb0 = rows_v[b, r0 + K // 2, pl.ds(c0, 16)]

                        def kstep(kk, acc):
                            ga = rows_v[b, r0 + kk, pl.ds(c0, 16)]
                            gb = rows_v[b, r0 + K // 2 + kk, pl.ds(c0, 16)]
                            return (jnp.maximum(acc[0], ga),
                                    jnp.maximum(acc[1], gb),
                                    acc[2] + (ga + gb),
                                    acc[3] + (ga * ga + gb * gb))

                        mxa, mxb, s1, s2 = lax.fori_loop(
                            1, K // 2, kstep,
                            (ga0, gb0, ga0 + gb0, ga0 * ga0 + gb0 * gb0))
                        mx = jnp.maximum(mxa, mxb)
                        m_v[j * CH + i, pl.ds(c0, 16)] = mx
                        zr = z_v[i, pl.ds(c0, 16)]
                        p_v[0, pl.ds(c0, 16)] += K * zr + s1
                        p_v[1, pl.ds(c0, 16)] += (
                            K * zr * zr + 2.0 * zr * s1 + s2)
            return 0

        lax.fori_loop(0, nch // 2, outer, 0)
        pltpu.sync_copy(m_v, m_hbm.at[pl.ds(base_node, nodes_pw)])
        pltpu.sync_copy(p_v, part_hbm.at[wid])

    return k(y, idxflat, z)


def _sc_ppf_gather(tbl8, ballflat):
    rows_pw = NK // NW
    mesh = plsc.VectorSubcoreMesh(core_axis_name="c", subcore_axis_name="s")

    chunk = 256
    nch = rows_pw // chunk

    @functools.partial(
        pl.kernel,
        mesh=mesh,
        out_type=jax.ShapeDtypeStruct((NK, 128), jnp.float32),
        scratch_types=[
            pltpu.VMEM((rows_pw,), jnp.int32),
            pltpu.VMEM((2, chunk, 128), jnp.float32),
            pltpu.VMEM_SHARED((N, 128), jnp.float32),
            pltpu.SemaphoreType.DMA,
        ],
    )
    def k(tbl_hbm, idx_hbm, out_hbm, idx_v, rows_v, sh, sem):
        sid = lax.axis_index("s")
        wid = sid * 2 + lax.axis_index("c")
        base = wid * rows_pw
        stage = N // 16
        so = pl.multiple_of(sid * stage, stage)
        pltpu.sync_copy(tbl_hbm.at[pl.ds(so, stage)], sh.at[pl.ds(so, stage)])
        pltpu.sync_copy(idx_hbm.at[pl.ds(base, rows_pw)], idx_v)
        plsc.subcore_barrier()

        def gstart(j, b):
            off = pl.multiple_of(j * chunk, chunk)
            pltpu.async_copy(
                sh.at[idx_v.at[pl.ds(off, chunk)]], rows_v.at[b], sem)

        def gwait():
            pltpu.make_async_copy(
                sh.at[idx_v.at[pl.ds(0, chunk)]], rows_v.at[0],
                sem).wait()

        gstart(0, 0)
        for j in range(nch):
            if j + 1 < nch:
                gstart(j + 1, (j + 1) % 2)
            gwait()
            pltpu.sync_copy(rows_v.at[j % 2],
                            out_hbm.at[pl.ds(base + j * chunk, chunk)])

    return k(tbl8, ballflat)


def _ppf_ang_body(gp_ref, pt_ref, nt_ref, w0_ref, f10_ref, st_ref):
    kstep = pl.program_id(0)
    g = gp_ref[...]
    gp = jnp.transpose(g)
    px, py, pz = pt_ref[0:1, :], pt_ref[1:2, :], pt_ref[2:3, :]
    nix, niy, niz = nt_ref[0:1, :], nt_ref[1:2, :], nt_ref[2:3, :]
    gx = gp[0:1, :] - px
    gy = gp[1:2, :] - py
    gz = gp[2:3, :] - pz
    njx, njy, njz = gp[3:4, :], gp[4:5, :], gp[5:6, :]

    def ang(ax, ay, az, bx, by, bz):
        cx = ay * bz - az * by
        cy = az * bx - ax * bz
        cz = ax * by - ay * bx
        yv = jnp.sqrt(cx * cx + cy * cy + cz * cz + 1e-12)
        xv = ax * bx + ay * by + az * bz
        return jnp.arctan2(yv, xv)

    nr_d = ang(nix, niy, niz, gx, gy, gz)
    ni_d = ang(njx, njy, njz, gx, gy, gz)
    nr_ni = ang(nix, niy, niz, njx, njy, njz)
    dn = jnp.sqrt(gx * gx + gy * gy + gz * gz + 1e-12)
    zr = jnp.zeros((6, N), jnp.float32)
    f10 = jnp.concatenate(
        [px, py, pz, gx, gy, gz, nr_d, ni_d, nr_ni, dn, zr], axis=0)
    f10_ref[...] = f10

    x0 = jnp.dot(w0_ref[...], f10, preferred_element_type=jnp.float32)

    @pl.when(kstep == 0)
    def _():
        st_ref[...] = jnp.zeros_like(st_ref)

    st_ref[0:1, 0:64] += jnp.sum(x0, axis=1)[None, :]
    st_ref[1:2, 0:64] += jnp.sum(x0 * x0, axis=1)[None, :]


def _ppf_ang(gplanes, pt, nt, w0p):
    return pl.pallas_call(
        _ppf_ang_body,
        grid=(K,),
        in_specs=[
            pl.BlockSpec((N, 128), lambda k: (k, 0)),
            pl.BlockSpec((4, N), lambda k: (0, 0)),
            pl.BlockSpec((4, N), lambda k: (0, 0)),
            pl.BlockSpec((64, 16), lambda k: (0, 0)),
        ],
        out_specs=[
            pl.BlockSpec((16, N), lambda k: (0, k)),
            pl.BlockSpec((8, 128), lambda k: (0, 0)),
        ],
        out_shape=[
            jax.ShapeDtypeStruct((16, NK), jnp.float32),
            jax.ShapeDtypeStruct((8, 128), jnp.float32),
        ],
    )(gplanes, pt, nt, w0p)


def _ppf_mid_body(f10_ref, w0_ref, w1_ref, m0_ref, i0_ref, st_ref):
    kstep = pl.program_id(0)
    x0 = jnp.dot(w0_ref[...], f10_ref[...], preferred_element_type=jnp.float32)
    h0 = (x0 - m0_ref[...]) * i0_ref[...]
    h0 = jnp.maximum(h0, 0.0)
    x1 = jnp.dot(w1_ref[...], h0, preferred_element_type=jnp.float32)

    @pl.when(kstep == 0)
    def _():
        st_ref[...] = jnp.zeros_like(st_ref)

    st_ref[0:1, :] += jnp.sum(x1, axis=1)[None, :]
    st_ref[1:2, :] += jnp.sum(x1 * x1, axis=1)[None, :]


def _ppf_mid(f10, w0p, w1, m0, i0):
    return pl.pallas_call(
        _ppf_mid_body,
        grid=(K,),
        in_specs=[
            pl.BlockSpec((16, N), lambda k: (0, k)),
            pl.BlockSpec((64, 16), lambda k: (0, 0)),
            pl.BlockSpec((128, 64), lambda k: (0, 0)),
            pl.BlockSpec((64, 1), lambda k: (0, 0)),
            pl.BlockSpec((64, 1), lambda k: (0, 0)),
        ],
        out_specs=pl.BlockSpec((8, 128), lambda k: (0, 0)),
        out_shape=jax.ShapeDtypeStruct((8, 128), jnp.float32),
    )(f10, w0p, w1, m0, i0)


def _ppf_last_body(f10_ref, w0_ref, w1_ref, w2_ref, m0_ref, i0_ref,
                   m1_ref, i1_ref, xmax_ref, st_ref):
    kstep = pl.program_id(0)
    x0 = jnp.dot(w0_ref[...], f10_ref[...], preferred_element_type=jnp.float32)
    h0 = jnp.maximum((x0 - m0_ref[...]) * i0_ref[...], 0.0)
    x1 = jnp.dot(w1_ref[...], h0, preferred_element_type=jnp.float32)
    h1 = jnp.maximum((x1 - m1_ref[...]) * i1_ref[...], 0.0)
    x2 = jnp.dot(w2_ref[...], h1, preferred_element_type=jnp.float32)

    @pl.when(kstep == 0)
    def _():
        xmax_ref[...] = x2
        st_ref[...] = jnp.zeros_like(st_ref)

    @pl.when(kstep > 0)
    def _():
        xmax_ref[...] = jnp.maximum(xmax_ref[...], x2)

    st_ref[0:1, 0:64] += jnp.sum(x2, axis=1)[None, :]
    st_ref[1:2, 0:64] += jnp.sum(x2 * x2, axis=1)[None, :]


def _ppf_last(f10, w0p, w1, w2, m0, i0, m1, i1):
    return pl.pallas_call(
        _ppf_last_body,
        grid=(K,),
        in_specs=[
            pl.BlockSpec((16, N), lambda k: (0, k)),
            pl.BlockSpec((64, 16), lambda k: (0, 0)),
            pl.BlockSpec((128, 64), lambda k: (0, 0)),
            pl.BlockSpec((64, 128), lambda k: (0, 0)),
            pl.BlockSpec((64, 1), lambda k: (0, 0)),
            pl.BlockSpec((64, 1), lambda k: (0, 0)),
            pl.BlockSpec((128, 1), lambda k: (0, 0)),
            pl.BlockSpec((128, 1), lambda k: (0, 0)),
        ],
        out_specs=[
            pl.BlockSpec((64, N), lambda k: (0, 0)),
            pl.BlockSpec((8, 128), lambda k: (0, 0)),
        ],
        out_shape=[
            jax.ShapeDtypeStruct((64, N), jnp.float32),
            jax.ShapeDtypeStruct((8, 128), jnp.float32),
        ],
    )(f10, w0p, w1, w2, m0, i0, m1, i1)


def _mm2_body(f_ref, wa_ref, wb_ref, ya_ref, yb_ref):
    f = f_ref[...]
    ya_ref[...] = jnp.dot(f, wa_ref[...], preferred_element_type=jnp.float32)
    yb_ref[...] = jnp.dot(f, wb_ref[...], preferred_element_type=jnp.float32)


def _mm2(f, wa, wb):
    Cin = f.shape[1]
    Ca, Cb = wa.shape[1], wb.shape[1]
    return pl.pallas_call(
        _mm2_body,
        grid=(N // 512,),
        in_specs=[
            pl.BlockSpec((512, Cin), lambda i: (i, 0)),
            pl.BlockSpec((Cin, Ca), lambda i: (0, 0)),
            pl.BlockSpec((Cin, Cb), lambda i: (0, 0)),
        ],
        out_specs=[
            pl.BlockSpec((512, Ca), lambda i: (i, 0)),
            pl.BlockSpec((512, Cb), lambda i: (i, 0)),
        ],
        out_shape=[
            jax.ShapeDtypeStruct((N, Ca), jnp.float32),
            jax.ShapeDtypeStruct((N, Cb), jnp.float32),
        ],
    )(f, wa, wb)


def _lrelu(x):
    return jnp.where(x >= 0, x, 0.2 * x)


def _gcnb_body(z_ref, mx_ref, mu_ref, iv_ref, wa_ref, wb_ref,
               f1_ref, ya_ref, yb_ref):
    f1 = _lrelu((z_ref[...] + mx_ref[...] - mu_ref[...]) * iv_ref[...])
    f1_ref[...] = f1
    ya_ref[...] = jnp.dot(f1, wa_ref[...], preferred_element_type=jnp.float32)
    yb_ref[...] = jnp.dot(f1, wb_ref[...], preferred_element_type=jnp.float32)


def _gcnb(z, mx, mu, iv, wa, wb):
    Cin = z.shape[1]
    Ca, Cb = wa.shape[1], wb.shape[1]
    return pl.pallas_call(
        _gcnb_body,
        grid=(N // 512,),
        in_specs=[
            pl.BlockSpec((512, Cin), lambda i: (i, 0)),
            pl.BlockSpec((512, Cin), lambda i: (i, 0)),
            pl.BlockSpec((1, Cin), lambda i: (0, 0)),
            pl.BlockSpec((1, Cin), lambda i: (0, 0)),
            pl.BlockSpec((Cin, Ca), lambda i: (0, 0)),
            pl.BlockSpec((Cin, Cb), lambda i: (0, 0)),
        ],
        out_specs=[
            pl.BlockSpec((512, Cin), lambda i: (i, 0)),
            pl.BlockSpec((512, Ca), lambda i: (i, 0)),
            pl.BlockSpec((512, Cb), lambda i: (i, 0)),
        ],
        out_shape=[
            jax.ShapeDtypeStruct((N, Cin), jnp.float32),
            jax.ShapeDtypeStruct((N, Ca), jnp.float32),
            jax.ShapeDtypeStruct((N, Cb), jnp.float32),
        ],
    )(z, mx, mu, iv, wa, wb)


def _gcnc_body(f_ref, f1_ref, z2_ref, mx2_ref, mu_ref, iv_ref, w3_ref,
               g3_ref, st_ref):
    i = pl.program_id(0)
    f2 = _lrelu((z2_ref[...] + mx2_ref[...] - mu_ref[...]) * iv_ref[...])
    f3 = jnp.concatenate([f_ref[...], f1_ref[...], f2], axis=1)
    g3 = jnp.dot(f3, w3_ref[...], preferred_element_type=jnp.float32)
    g3_ref[...] = g3

    @pl.when(i == 0)
    def _():
        st_ref[...] = jnp.zeros_like(st_ref)

    st_ref[0:1, :] += jnp.sum(g3, axis=0)[None, :]
    st_ref[1:2, :] += jnp.sum(g3 * g3, axis=0)[None, :]


def _gcnc(f, f1, z2, mx2, mu, iv, w3t):
    return pl.pallas_call(
        _gcnc_body,
        grid=(N // 512,),
        in_specs=[
            pl.BlockSpec((512, 128), lambda i: (i, 0)),
            pl.BlockSpec((512, 128), lambda i: (i, 0)),
            pl.BlockSpec((512, 256), lambda i: (i, 0)),
            pl.BlockSpec((512, 256), lambda i: (i, 0)),
            pl.BlockSpec((1, 256), lambda i: (0, 0)),
            pl.BlockSpec((1, 256), lambda i: (0, 0)),
            pl.BlockSpec((512, 128), lambda i: (0, 0)),
        ],
        out_specs=[
            pl.BlockSpec((512, 128), lambda i: (i, 0)),
            pl.BlockSpec((8, 128), lambda i: (0, 0)),
        ],
        out_shape=[
            jax.ShapeDtypeStruct((N, 128), jnp.float32),
            jax.ShapeDtypeStruct((8, 128), jnp.float32),
        ],
    )(f, f1, z2, mx2, mu, iv, w3t)


def _heada_body(g3_ref, xm_ref, m3_ref, i3_ref, mp_ref, ip_ref,
                w0_ref, b0_ref, h0_ref, st_ref):
    i = pl.program_id(0)
    fgcn = _lrelu((g3_ref[...] - m3_ref[...]) * i3_ref[...])
    fppf = jnp.maximum((xm_ref[...] - mp_ref[...]) * ip_ref[...], 0.0)
    h = jnp.concatenate([fppf, fgcn], axis=1)
    h0 = jnp.dot(h, w0_ref[...], preferred_element_type=jnp.float32) + b0_ref[...]
    h0_ref[...] = h0

    @pl.when(i == 0)
    def _():
        st_ref[...] = jnp.zeros_like(st_ref)

    st_ref[0:1, :] += jnp.sum(h0, axis=0)[None, :]
    st_ref[1:2, :] += jnp.sum(h0 * h0, axis=0)[None, :]


def _heada(g3, xmt, m3, i3, mp, ip, w0t, b0):
    return pl.pallas_call(
        _heada_body,
        grid=(N // 512,),
        in_specs=[
            pl.BlockSpec((512, 128), lambda i: (i, 0)),
            pl.BlockSpec((512, 64), lambda i: (i, 0)),
            pl.BlockSpec((1, 128), lambda i: (0, 0)),
            pl.BlockSpec((1, 128), lambda i: (0, 0)),
            pl.BlockSpec((1, 64), lambda i: (0, 0)),
            pl.BlockSpec((1, 64), lambda i: (0, 0)),
            pl.BlockSpec((192, 192), lambda i: (0, 0)),
            pl.BlockSpec((1, 192), lambda i: (0, 0)),
        ],
        out_specs=[
            pl.BlockSpec((512, 192), lambda i: (i, 0)),
            pl.BlockSpec((8, 192), lambda i: (0, 0)),
        ],
        out_shape=[
            jax.ShapeDtypeStruct((N, 192), jnp.float32),
            jax.ShapeDtypeStruct((8, 192), jnp.float32),
        ],
    )(g3, xmt, m3, i3, mp, ip, w0t, b0)


def _headb_body(h0_ref, m_ref, iv_ref, w1_ref, b1_ref, h1_ref, st_ref):
    i = pl.program_id(0)
    a0 = _lrelu((h0_ref[...] - m_ref[...]) * iv_ref[...])
    h1 = jnp.dot(a0, w1_ref[...], preferred_element_type=jnp.float32) + b1_ref[...]
    h1_ref[...] = h1

    @pl.when(i == 0)
    def _():
        st_ref[...] = jnp.zeros_like(st_ref)

    st_ref[0:1, :] += jnp.sum(h1, axis=0)[None, :]
    st_ref[1:2, :] += jnp.sum(h1 * h1, axis=0)[None, :]


def _headb(h0, m, iv, w1t, b1):
    return pl.pallas_call(
        _headb_body,
        grid=(N // 512,),
        in_specs=[
            pl.BlockSpec((512, 192), lambda i: (i, 0)),
            pl.BlockSpec((1, 192), lambda i: (0, 0)),
            pl.BlockSpec((1, 192), lambda i: (0, 0)),
            pl.BlockSpec((192, 128), lambda i: (0, 0)),
            pl.BlockSpec((1, 128), lambda i: (0, 0)),
        ],
        out_specs=[
            pl.BlockSpec((512, 128), lambda i: (i, 0)),
            pl.BlockSpec((8, 128), lambda i: (0, 0)),
        ],
        out_shape=[
            jax.ShapeDtypeStruct((N, 128), jnp.float32),
            jax.ShapeDtypeStruct((8, 128), jnp.float32),
        ],
    )(h0, m, iv, w1t, b1)


def _headc_body(h1_ref, m_ref, iv_ref, o_ref):
    o_ref[...] = _lrelu((h1_ref[...] - m_ref[...]) * iv_ref[...])


def _headc(h1, m, iv):
    return pl.pallas_call(
        _headc_body,
        grid=(N // 512,),
        in_specs=[
            pl.BlockSpec((512, 128), lambda i: (i, 0)),
            pl.BlockSpec((1, 128), lambda i: (0, 0)),
            pl.BlockSpec((1, 128), lambda i: (0, 0)),
        ],
        out_specs=pl.BlockSpec((512, 128), lambda i: (i, 0)),
        out_shape=jax.ShapeDtypeStruct((N, 128), jnp.float32),
    )(h1, m, iv)


def _mi(st, count, C, rowvec):
    s = st[0, :C]
    sq = st[1, :C]
    m = s / count
    v = sq / count - m * m
    iv = lax.rsqrt(v + EPS)
    if rowvec:
        return m[None, :], iv[None, :]
    return m[:, None], iv[:, None]


def kernel(coords, feats, normals, gcn_w1, gcn_w2, gcn_w3, ppf_w0, ppf_w1,
           ppf_w2, fused_w0, fused_b0, fused_w1, fused_b1):
    P = coords[0].T
    F = feats[0].T
    Nm = normals[0].T

    ppad = jnp.pad(P, ((0, 0), (0, 5)))
    ppad_t = ppad.T
    sel = _select(ppad, ppad_t)
    knn_flat = sel[:, 1:K + 1].reshape(NK)
    ball_flat = sel[:, 64:64 + K].T.reshape(NK)

    wc1, wn1 = gcn_w1[:, :128], gcn_w1[:, 128:]
    y1, z1 = _mm2(F, wn1.T, (wc1 - wn1).T)
    mx1, parts1 = _sc_ec_reduce(y1, knn_flat, z1)
    tbl8 = jnp.pad(jnp.concatenate([P, Nm], axis=1), ((0, 0), (0, 122)))
    gplanes = _sc_ppf_gather(tbl8, ball_flat)

    pt4 = jnp.pad(P.T, ((0, 1), (0, 0)))
    nt4 = jnp.pad(Nm.T, ((0, 1), (0, 0)))
    w0p = jnp.pad(ppf_w0, ((0, 0), (0, 6)))
    f10, st0 = _ppf_ang(gplanes, pt4, nt4, w0p)
    m0, i0 = _mi(st0, NK, 64, rowvec=False)

    mu1, iv1 = _mi(jnp.sum(parts1, axis=0), NK, 128, rowvec=True)
    wc2, wn2 = gcn_w2[:, :128], gcn_w2[:, 128:]
    f1, y2, z2 = _gcnb(z1, mx1, mu1, iv1, wn2.T, (wc2 - wn2).T)
    mx2a, parts2a = _sc_ec_reduce(y2[:, :128], knn_flat, z2[:, :128])
    mx2b, parts2b = _sc_ec_reduce(y2[:, 128:], knn_flat, z2[:, 128:],
                                  use_spmem=False)

    st1 = _ppf_mid(f10, w0p, ppf_w1, m0, i0)
    m1, i1 = _mi(st1, NK, 128, rowvec=False)
    xmax, st2 = _ppf_last(f10, w0p, ppf_w1, ppf_w2, m0, i0, m1, i1)
    mp, ip = _mi(st2, NK, 64, rowvec=True)

    mx2 = jnp.concatenate([mx2a, mx2b], axis=1)
    st2g = jnp.concatenate(
        [jnp.sum(parts2a, axis=0), jnp.sum(parts2b, axis=0)], axis=1)
    mu2, iv2 = _mi(st2g, NK, 256, rowvec=True)

    g3, st3 = _gcnc(F, f1, z2, mx2, mu2, iv2, gcn_w3.T)
    m3, i3 = _mi(st3, N, 128, rowvec=True)

    h0, stH0 = _heada(g3, xmax.T, m3, i3, mp, ip, fused_w0.T,
                      fused_b0[None, :])
    mh0, ih0 = _mi(stH0, N, 192, rowvec=True)
    h1, stH1 = _headb(h0, mh0, ih0, fused_w1.T, fused_b1[None, :])
    mh1, ih1 = _mi(stH1, N, 128, rowvec=True)
    out = _headc(h1, mh1, ih1)
    return out.T[None]
